# Initial kernel scaffold; baseline (speedup 1.0000x reference)
import jax
import jax.numpy as jnp
from jax.experimental import pallas as pl


def kernel(item_fea, W_publisher, W_author, W_year, W_iid, W_title):
    raise NotImplementedError("write your pallas kernel here")


# speedup vs baseline: 1.8923x; 1.8923x over previous
import functools

import jax
import jax.numpy as jnp
from jax import lax
from jax.experimental import pallas as pl
from jax.experimental.pallas import tpu as pltpu
from jax.experimental.pallas import tpu_sc as plsc

B = 16384
EMB = 32
NUM_AUTHOR = 4211
NUM_PUBLISHER = 716
NFEA = 5

NC = 2
NS = 16
NW = NC * NS
IPW = B // NW
RPW = 2 * IPW
FPW = NFEA * IPW
ICHUNK = 128
NCHUNK = RPW // ICHUNK


def _build_sc_call():
    mesh = plsc.VectorSubcoreMesh(core_axis_name="c", subcore_axis_name="s")

    @functools.partial(
        pl.kernel,
        mesh=mesh,
        compiler_params=pltpu.CompilerParams(
            needs_layout_passes=False, use_tc_tiling_on_sc=False
        ),
        out_type=jax.ShapeDtypeStruct((2 * B, EMB), jnp.float32),
        scratch_types=[
            pltpu.VMEM((FPW,), jnp.int32),
            pltpu.VMEM((NCHUNK, ICHUNK), jnp.int32),
            pltpu.VMEM((RPW, EMB), jnp.float32),
            pltpu.SemaphoreType.DMA,
        ],
    )
    def sc_kernel(fea_hbm, tab_hbm, out_hbm, fea_v, ci_v, rows_v, sem):
        wid = lax.axis_index("s") * NC + lax.axis_index("c")

        pltpu.sync_copy(fea_hbm.at[pl.ds(wid * FPW, FPW)], fea_v)

        i16 = lax.iota(jnp.int32, 16)
        half = lax.shift_right_logical(i16, 1)
        odd = lax.bitwise_and(i16, jnp.int32(1))
        base_idx = half * NFEA + 2 * odd + 2
        offset = odd * NUM_AUTHOR
        for m in range(RPW // 16):
            idx = base_idx + jnp.int32(8 * NFEA * m)
            v = plsc.load_gather(fea_v, [idx]) + offset
            ci_v[m // (ICHUNK // 16), pl.ds((m % (ICHUNK // 16)) * 16, 16)] = v

        copies = [
            pltpu.async_copy(
                tab_hbm.at[ci_v.at[j]],
                rows_v.at[pl.ds(j * ICHUNK, ICHUNK)],
                sem,
            )
            for j in range(NCHUNK)
        ]
        for c in copies:
            c.wait()

        pltpu.sync_copy(rows_v, out_hbm.at[pl.ds(wid * RPW, RPW)])

    return sc_kernel


def kernel(item_fea, W_publisher, W_author, W_year, W_iid, W_title):
    fea = item_fea.astype(jnp.int32).reshape(-1)
    tab = jnp.concatenate([W_author, W_publisher], axis=0)
    out = _build_sc_call()(fea, tab)
    return out.reshape(B, 2 * EMB)
